# baseline (speedup 1.0000x reference)
import functools
import math

import jax
import jax.numpy as jnp
from jax import lax
from jax.experimental import pallas as pl
from jax.experimental.pallas import tpu as pltpu
from jax.experimental.pallas import tpu_sc as plsc

_NC, _NS, _L = 2, 16, 16
_NW = _NC * _NS


def _route_body(C, E, NCH, CHA, x_ref, wr_ref, br_ref,
                dest_ref, s0_ref, s1_ref, w0_ref, w1_ref, xb_ref):
    N, D = x_ref.shape
    EP = E
    xv = x_ref[:]
    u = lax.bitcast_convert_type(xv, jnp.uint32)
    r = (u + jnp.uint32(0x8000)) >> 16
    word = r[:, :D // 2] | (r[:, D // 2:] << 16)
    xb_ref[:] = lax.bitcast_convert_type(word, jnp.int32)
    logits = jnp.dot(xv, wr_ref[:],
                     preferred_element_type=jnp.float32) + br_ref[:]
    lane = lax.broadcasted_iota(jnp.int32, (N, EP), 1)
    m0 = jnp.max(logits, axis=1, keepdims=True)
    a0 = jnp.min(jnp.where(logits == m0, lane, 127), axis=1, keepdims=True)
    l2 = jnp.where(lane == a0, -1e30, logits)
    m1 = jnp.max(l2, axis=1, keepdims=True)
    a1 = jnp.min(jnp.where(l2 == m1, lane, 127), axis=1, keepdims=True)
    g0 = 1.0 / (1.0 + jnp.exp(m1 - m0))
    g1 = 1.0 - g0
    oh0 = (lane == a0).astype(jnp.float32)
    oh1 = (lane == a1).astype(jnp.float32)

    R = 512
    rr = lax.broadcasted_iota(jnp.int32, (R, R), 0)
    cc = lax.broadcasted_iota(jnp.int32, (R, R), 1)
    tstrict = (cc < rr).astype(jnp.float32)

    def excl_cumsum(oh, carry):
        parts = []
        for c in range(N // R):
            blk = oh[c * R:(c + 1) * R, :]
            parts.append(jnp.dot(tstrict, blk,
                                 preferred_element_type=jnp.float32) + carry)
            carry = carry + jnp.sum(blk, axis=0, keepdims=True)
        return jnp.concatenate(parts, axis=0), carry

    zero = jnp.zeros((1, EP), jnp.float32)
    p0, tot0 = excl_cumsum(oh0, zero)
    p1, _ = excl_cumsum(oh1, tot0)
    pos0 = jnp.sum(p0 * oh0, axis=1, keepdims=True).astype(jnp.int32)
    pos1 = jnp.sum(p1 * oh1, axis=1, keepdims=True).astype(jnp.int32)
    v0 = pos0 < C
    v1 = pos1 < C
    gv0 = jnp.where(v0, g0, 0.0)
    gv1 = jnp.where(v1, g1, 0.0)
    den = jnp.maximum(gv0 + gv1, 1e-8)
    slot0 = a0 * C + pos0
    slot1 = a1 * C + pos1
    dump = E * C
    half = N // (CHA * NCH)
    dest0 = jnp.where(v0, slot0, dump).reshape(half, NCH, CHA)
    dest1 = jnp.where(v1, slot1, dump).reshape(half, NCH, CHA)
    dest_ref[:] = jnp.concatenate([dest0, dest1], axis=0)
    s0_ref[:] = jnp.where(v0, slot0, 0).reshape(1, N)
    s1_ref[:] = jnp.where(v1, slot1, 0).reshape(1, N)
    w0_ref[:] = (gv0 / den).reshape(1, N)
    w1_ref[:] = (gv1 / den).reshape(1, N)


def _ffn_body(DH, xe_ref, gamma_ref, beta_ref, w1_ref, b1_ref, w2_ref,
              b2_ref, out_ref):
    wv = lax.bitcast_convert_type(xe_ref[:], jnp.uint32)
    xlo = lax.bitcast_convert_type(wv << 16, jnp.float32)
    xhi = lax.bitcast_convert_type(wv & jnp.uint32(0xFFFF0000), jnp.float32)
    xb = jnp.concatenate([xlo, xhi], axis=1)
    mu = jnp.mean(xb, axis=1, keepdims=True)
    xc = xb - mu
    var = jnp.mean(xc * xc, axis=1, keepdims=True)
    xn = xc * lax.rsqrt(var + 1e-5)
    xn = xn * gamma_ref[0] + beta_ref[0]
    pre = jnp.dot(xn.astype(jnp.bfloat16), w1_ref[0].astype(jnp.bfloat16),
                  preferred_element_type=jnp.float32) + b1_ref[0]
    a = pre[:, :DH]
    g = pre[:, DH:]
    act = a * (1.0 / (1.0 + jnp.exp(-g)))
    out_ref[:] = jnp.dot(act.astype(jnp.bfloat16),
                         w2_ref[0].astype(jnp.bfloat16),
                         preferred_element_type=jnp.float32) + b2_ref[0]


def _dispatch_body(N, NCH, CHA, x_hbm, dest_hbm, xe_hbm, didx_v,
                   rows0, rows1, semr0, semr1, semw0, semw1):
    wid = lax.axis_index("s") * _NC + lax.axis_index("c")
    tok0 = (wid % (_NW // 2)) * (2 * N // _NW)
    pltpu.sync_copy(dest_hbm.at[wid], didx_v)
    rows = [rows0, rows1]
    semr = [semr0, semr1]
    semw = [semw0, semw1]
    rd = [None, None]
    wr = [None, None]
    for b in range(min(2, NCH)):
        rd[b] = pltpu.async_copy(
            x_hbm.at[pl.ds(tok0 + b * CHA, CHA)], rows[b], semr[b])
    for c in range(NCH):
        b = c % 2
        rd[b].wait()
        wr[b] = pltpu.async_copy(rows[b], xe_hbm.at[didx_v.at[c]], semw[b])
        if c + 2 < NCH:
            wr[b].wait()
            rd[b] = pltpu.async_copy(
                x_hbm.at[pl.ds(tok0 + (c + 2) * CHA, CHA)], rows[b], semr[b])
    for c in range(max(0, NCH - 2), NCH):
        wr[c % 2].wait()


def _combine_body(D, TOK, TCH, o_hbm, s0_hbm, s1_hbm, w0_hbm, w1_hbm, y_hbm,
                  s0_v, s1_v, w0_v, w1_v, bufa0, bufa1, bufb0, bufb1,
                  ybuf0, ybuf1, sema0, sema1, semb0, semb1, semy0, semy1):
    NCHK = TOK // TCH
    wid = lax.axis_index("s") * _NC + lax.axis_index("c")
    tb = wid * TOK
    pltpu.sync_copy(s0_hbm.at[0, pl.ds(tb, TOK)], s0_v)
    pltpu.sync_copy(s1_hbm.at[0, pl.ds(tb, TOK)], s1_v)
    pltpu.sync_copy(w0_hbm.at[0, pl.ds(tb, TOK)], w0_v)
    pltpu.sync_copy(w1_hbm.at[0, pl.ds(tb, TOK)], w1_v)
    bufa = [bufa0, bufa1]
    bufb = [bufb0, bufb1]
    ybuf = [ybuf0, ybuf1]
    sema = [sema0, sema1]
    semb = [semb0, semb1]
    semy = [semy0, semy1]
    ga = [None, None]
    gb = [None, None]
    wy = [None, None]
    for b in range(min(2, NCHK)):
        ga[b] = pltpu.async_copy(
            o_hbm.at[s0_v.at[pl.ds(b * TCH, TCH)]], bufa[b], sema[b])
        gb[b] = pltpu.async_copy(
            o_hbm.at[s1_v.at[pl.ds(b * TCH, TCH)]], bufb[b], semb[b])
    for ci in range(NCHK):
        b = ci % 2
        off = ci * TCH
        ga[b].wait()
        gb[b].wait()
        if wy[b] is not None:
            wy[b].wait()

        def tbody(t, c):
            ti = off + t
            wa = plsc.load_gather(w0_v, [jnp.full((16,), ti, jnp.int32)])
            wb = plsc.load_gather(w1_v, [jnp.full((16,), ti, jnp.int32)])
            zero = jnp.zeros((16,), jnp.float32)
            ya, yb, yy = bufa[b], bufb[b], ybuf[b]
            for v in range(D // 16):
                sl = pl.ds(v * 16, 16)
                yy[t, sl] = (jnp.where(wa > 0, wa * ya[t, sl], zero)
                             + jnp.where(wb > 0, wb * yb[t, sl], zero))
            return c
        lax.fori_loop(0, TCH, tbody, 0)
        wy[b] = pltpu.async_copy(
            ybuf[b], y_hbm.at[pl.ds(tb + off, TCH)], semy[b])
        if ci + 2 < NCHK:
            off2 = (ci + 2) * TCH
            ga[b] = pltpu.async_copy(
                o_hbm.at[s0_v.at[pl.ds(off2, TCH)]], bufa[b], sema[b])
            gb[b] = pltpu.async_copy(
                o_hbm.at[s1_v.at[pl.ds(off2, TCH)]], bufb[b], semb[b])
    for ci in range(max(0, NCHK - 2), NCHK):
        wy[ci % 2].wait()


def kernel(h, Wr, br, gamma, beta, W1, b1, W2, b2):
    B, T, D = h.shape
    N = B * T
    E = Wr.shape[1]
    DH = W2.shape[1]
    K = 2
    C = math.ceil(1.25 * (N * K) / E)
    SLOTS = E * C
    assert SLOTS % _NW == 0 and N % _NW == 0 and D % _L == 0
    APW = 2 * N // _NW
    NCH = 4
    CHA = APW // NCH
    TOK = N // _NW
    TCH = TOK // 8

    x = h.reshape(N, D)

    route = pl.pallas_call(
        functools.partial(_route_body, C, E, NCH, CHA),
        out_shape=[jax.ShapeDtypeStruct((_NW, NCH, CHA), jnp.int32),
                   jax.ShapeDtypeStruct((1, N), jnp.int32),
                   jax.ShapeDtypeStruct((1, N), jnp.int32),
                   jax.ShapeDtypeStruct((1, N), jnp.float32),
                   jax.ShapeDtypeStruct((1, N), jnp.float32),
                   jax.ShapeDtypeStruct((N, D // 2), jnp.int32)],
    )
    dest_all, s0, s1, w0, w1, xb16 = route(x, Wr, br.reshape(1, E))

    sc_params = pltpu.CompilerParams(needs_layout_passes=False)
    mesh = plsc.VectorSubcoreMesh(core_axis_name="c", subcore_axis_name="s")
    dispatch = pl.kernel(
        functools.partial(_dispatch_body, N, NCH, CHA),
        mesh=mesh,
        compiler_params=sc_params,
        out_type=jax.ShapeDtypeStruct((SLOTS + 8, D // 2), jnp.int32),
        scratch_types=[
            pltpu.VMEM((NCH, CHA), jnp.int32),
            pltpu.VMEM((CHA, D // 2), jnp.int32),
            pltpu.VMEM((CHA, D // 2), jnp.int32),
            pltpu.SemaphoreType.DMA,
            pltpu.SemaphoreType.DMA,
            pltpu.SemaphoreType.DMA,
            pltpu.SemaphoreType.DMA,
        ],
    )
    xe = dispatch(xb16, dest_all)

    ffn = pl.pallas_call(
        functools.partial(_ffn_body, DH),
        grid=(E,),
        in_specs=[
            pl.BlockSpec((C, D // 2), lambda e: (e, 0)),
            pl.BlockSpec((1, 1, D), lambda e: (e, 0, 0)),
            pl.BlockSpec((1, 1, D), lambda e: (e, 0, 0)),
            pl.BlockSpec((1, D, 2 * DH), lambda e: (e, 0, 0)),
            pl.BlockSpec((1, 1, 2 * DH), lambda e: (e, 0, 0)),
            pl.BlockSpec((1, DH, D), lambda e: (e, 0, 0)),
            pl.BlockSpec((1, 1, D), lambda e: (e, 0, 0)),
        ],
        out_specs=pl.BlockSpec((C, D), lambda e: (e, 0)),
        out_shape=jax.ShapeDtypeStruct((SLOTS, D), jnp.float32),
    )
    oexp = ffn(xe, gamma.reshape(E, 1, D), beta.reshape(E, 1, D), W1,
               b1.reshape(E, 1, 2 * DH), W2, b2.reshape(E, 1, D))

    combine = pl.kernel(
        functools.partial(_combine_body, D, TOK, TCH),
        mesh=plsc.VectorSubcoreMesh(core_axis_name="c", subcore_axis_name="s"),
        compiler_params=sc_params,
        out_type=jax.ShapeDtypeStruct((N, D), jnp.float32),
        scratch_types=[
            pltpu.VMEM((TOK,), jnp.int32),
            pltpu.VMEM((TOK,), jnp.int32),
            pltpu.VMEM((TOK,), jnp.float32),
            pltpu.VMEM((TOK,), jnp.float32),
            pltpu.VMEM((TCH, D), jnp.float32),
            pltpu.VMEM((TCH, D), jnp.float32),
            pltpu.VMEM((TCH, D), jnp.float32),
            pltpu.VMEM((TCH, D), jnp.float32),
            pltpu.VMEM((TCH, D), jnp.float32),
            pltpu.VMEM((TCH, D), jnp.float32),
            pltpu.SemaphoreType.DMA,
            pltpu.SemaphoreType.DMA,
            pltpu.SemaphoreType.DMA,
            pltpu.SemaphoreType.DMA,
            pltpu.SemaphoreType.DMA,
            pltpu.SemaphoreType.DMA,
        ],
    )
    y = combine(oexp, s0, s1, w0, w1)
    return y.reshape(B, T, D)


# speedup vs baseline: 1.1791x; 1.1791x over previous
import functools
import math

import jax
import jax.numpy as jnp
from jax import lax
from jax.experimental import pallas as pl
from jax.experimental.pallas import tpu as pltpu
from jax.experimental.pallas import tpu_sc as plsc

_NC, _NS, _L = 2, 16, 16
_NW = _NC * _NS


def _route_body(C, E, NCH, CHA, x_ref, wr_ref, br_ref,
                dest_ref, s0_ref, s1_ref, w0_ref, w1_ref, xb_ref):
    N, D = x_ref.shape
    EP = E
    xv = x_ref[:]
    u = lax.bitcast_convert_type(xv, jnp.uint32)
    r = (u + jnp.uint32(0x8000)) >> 16
    word = r[:, :D // 2] | (r[:, D // 2:] << 16)
    xb_ref[:] = lax.bitcast_convert_type(word, jnp.int32)
    logits = jnp.dot(xv, wr_ref[:],
                     preferred_element_type=jnp.float32) + br_ref[:]
    lane = lax.broadcasted_iota(jnp.int32, (N, EP), 1)
    m0 = jnp.max(logits, axis=1, keepdims=True)
    a0 = jnp.min(jnp.where(logits == m0, lane, 127), axis=1, keepdims=True)
    l2 = jnp.where(lane == a0, -1e30, logits)
    m1 = jnp.max(l2, axis=1, keepdims=True)
    a1 = jnp.min(jnp.where(l2 == m1, lane, 127), axis=1, keepdims=True)
    g0 = 1.0 / (1.0 + jnp.exp(m1 - m0))
    g1 = 1.0 - g0
    oh0 = (lane == a0).astype(jnp.float32)
    oh1 = (lane == a1).astype(jnp.float32)

    R = 512
    rr = lax.broadcasted_iota(jnp.int32, (R, R), 0)
    cc = lax.broadcasted_iota(jnp.int32, (R, R), 1)
    tstrict = (cc < rr).astype(jnp.float32)

    def excl_cumsum(oh, carry):
        parts = []
        for c in range(N // R):
            blk = oh[c * R:(c + 1) * R, :]
            parts.append(jnp.dot(tstrict, blk,
                                 preferred_element_type=jnp.float32) + carry)
            carry = carry + jnp.sum(blk, axis=0, keepdims=True)
        return jnp.concatenate(parts, axis=0), carry

    zero = jnp.zeros((1, EP), jnp.float32)
    p0, tot0 = excl_cumsum(oh0, zero)
    p1, _ = excl_cumsum(oh1, tot0)
    pos0 = jnp.sum(p0 * oh0, axis=1, keepdims=True).astype(jnp.int32)
    pos1 = jnp.sum(p1 * oh1, axis=1, keepdims=True).astype(jnp.int32)
    v0 = pos0 < C
    v1 = pos1 < C
    gv0 = jnp.where(v0, g0, 0.0)
    gv1 = jnp.where(v1, g1, 0.0)
    den = jnp.maximum(gv0 + gv1, 1e-8)
    slot0 = a0 * C + pos0
    slot1 = a1 * C + pos1
    dump = E * C
    half = N // (CHA * NCH)
    dest0 = jnp.where(v0, slot0, dump).reshape(half, NCH, CHA)
    dest1 = jnp.where(v1, slot1, dump).reshape(half, NCH, CHA)
    dest_ref[:] = jnp.concatenate([dest0, dest1], axis=0)
    s0_ref[:] = jnp.where(v0, slot0, 0).reshape(1, N)
    s1_ref[:] = jnp.where(v1, slot1, 0).reshape(1, N)
    w0_ref[:] = (gv0 / den).reshape(1, N)
    w1_ref[:] = (gv1 / den).reshape(1, N)


def _ffn_body(DH, xe_ref, gamma_ref, beta_ref, w1_ref, b1_ref, w2_ref,
              b2_ref, out_ref):
    wv = lax.bitcast_convert_type(xe_ref[:], jnp.uint32)
    xlo = lax.bitcast_convert_type(wv << 16, jnp.float32)
    xhi = lax.bitcast_convert_type(wv & jnp.uint32(0xFFFF0000), jnp.float32)
    xb = jnp.concatenate([xlo, xhi], axis=1)
    mu = jnp.mean(xb, axis=1, keepdims=True)
    xc = xb - mu
    var = jnp.mean(xc * xc, axis=1, keepdims=True)
    xn = xc * lax.rsqrt(var + 1e-5)
    xn = xn * gamma_ref[0] + beta_ref[0]
    pre = jnp.dot(xn.astype(jnp.bfloat16), w1_ref[0].astype(jnp.bfloat16),
                  preferred_element_type=jnp.float32) + b1_ref[0]
    a = pre[:, :DH]
    g = pre[:, DH:]
    act = a * (1.0 / (1.0 + jnp.exp(-g)))
    out_ref[:] = jnp.dot(act.astype(jnp.bfloat16),
                         w2_ref[0].astype(jnp.bfloat16),
                         preferred_element_type=jnp.float32) + b2_ref[0]


def _dispatch_body(N, NCH, CHA, x_hbm, dest_hbm, xe_hbm, didx_v,
                   rows0, rows1, semr0, semr1, semw0, semw1):
    wid = lax.axis_index("s") * _NC + lax.axis_index("c")
    tok0 = (wid % (_NW // 2)) * (2 * N // _NW)
    pltpu.sync_copy(dest_hbm.at[wid], didx_v)
    rows = [rows0, rows1]
    semr = [semr0, semr1]
    semw = [semw0, semw1]
    rd = [None, None]
    wr = [None, None]
    for b in range(min(2, NCH)):
        rd[b] = pltpu.async_copy(
            x_hbm.at[pl.ds(tok0 + b * CHA, CHA)], rows[b], semr[b])
    for c in range(NCH):
        b = c % 2
        rd[b].wait()
        wr[b] = pltpu.async_copy(rows[b], xe_hbm.at[didx_v.at[c]], semw[b])
        if c + 2 < NCH:
            wr[b].wait()
            rd[b] = pltpu.async_copy(
                x_hbm.at[pl.ds(tok0 + (c + 2) * CHA, CHA)], rows[b], semr[b])
    for c in range(max(0, NCH - 2), NCH):
        wr[c % 2].wait()


def _combine_body(D, TOK, TCH, o_hbm, s0_hbm, s1_hbm, w0_hbm, w1_hbm, y_hbm,
                  s0_v, s1_v, w0_v, w1_v, bufa0, bufa1, bufb0, bufb1,
                  ybuf0, ybuf1, sema0, sema1, semb0, semb1, semy0, semy1):
    NCHK = TOK // TCH
    wid = lax.axis_index("s") * _NC + lax.axis_index("c")
    tb = wid * TOK
    pltpu.sync_copy(s0_hbm.at[0, pl.ds(tb, TOK)], s0_v)
    pltpu.sync_copy(s1_hbm.at[0, pl.ds(tb, TOK)], s1_v)
    pltpu.sync_copy(w0_hbm.at[0, pl.ds(tb, TOK)], w0_v)
    pltpu.sync_copy(w1_hbm.at[0, pl.ds(tb, TOK)], w1_v)
    bufa = [bufa0, bufa1]
    bufb = [bufb0, bufb1]
    ybuf = [ybuf0, ybuf1]
    sema = [sema0, sema1]
    semb = [semb0, semb1]
    semy = [semy0, semy1]
    ga = [None, None]
    gb = [None, None]
    wy = [None, None]
    for b in range(min(2, NCHK)):
        ga[b] = pltpu.async_copy(
            o_hbm.at[s0_v.at[pl.ds(b * TCH, TCH)]], bufa[b], sema[b])
        gb[b] = pltpu.async_copy(
            o_hbm.at[s1_v.at[pl.ds(b * TCH, TCH)]], bufb[b], semb[b])
    for ci in range(NCHK):
        b = ci % 2
        off = ci * TCH
        ga[b].wait()
        gb[b].wait()
        if wy[b] is not None:
            wy[b].wait()

        def tbody(t, c):
            ti = off + t
            wa = plsc.load_gather(w0_v, [jnp.full((16,), ti, jnp.int32)])
            wb = plsc.load_gather(w1_v, [jnp.full((16,), ti, jnp.int32)])
            zero = jnp.zeros((16,), jnp.float32)
            ya, yb, yy = bufa[b], bufb[b], ybuf[b]
            for v in range(D // 16):
                sl = pl.ds(v * 16, 16)
                yy[t, sl] = (jnp.where(wa > 0, wa * ya[t, sl], zero)
                             + jnp.where(wb > 0, wb * yb[t, sl], zero))
            return c
        lax.fori_loop(0, TCH, tbody, 0)
        wy[b] = pltpu.async_copy(
            ybuf[b], y_hbm.at[pl.ds(tb + off, TCH)], semy[b])
        if ci + 2 < NCHK:
            off2 = (ci + 2) * TCH
            ga[b] = pltpu.async_copy(
                o_hbm.at[s0_v.at[pl.ds(off2, TCH)]], bufa[b], sema[b])
            gb[b] = pltpu.async_copy(
                o_hbm.at[s1_v.at[pl.ds(off2, TCH)]], bufb[b], semb[b])
    for ci in range(max(0, NCHK - 2), NCHK):
        wy[ci % 2].wait()


def kernel(h, Wr, br, gamma, beta, W1, b1, W2, b2):
    B, T, D = h.shape
    N = B * T
    E = Wr.shape[1]
    DH = W2.shape[1]
    K = 2
    C = math.ceil(1.25 * (N * K) / E)
    SLOTS = E * C
    assert SLOTS % _NW == 0 and N % _NW == 0 and D % _L == 0
    APW = 2 * N // _NW
    NCH = 4
    CHA = APW // NCH
    TOK = N // _NW
    TCH = TOK // 4

    x = h.reshape(N, D)

    route = pl.pallas_call(
        functools.partial(_route_body, C, E, NCH, CHA),
        out_shape=[jax.ShapeDtypeStruct((_NW, NCH, CHA), jnp.int32),
                   jax.ShapeDtypeStruct((1, N), jnp.int32),
                   jax.ShapeDtypeStruct((1, N), jnp.int32),
                   jax.ShapeDtypeStruct((1, N), jnp.float32),
                   jax.ShapeDtypeStruct((1, N), jnp.float32),
                   jax.ShapeDtypeStruct((N, D // 2), jnp.int32)],
    )
    dest_all, s0, s1, w0, w1, xb16 = route(x, Wr, br.reshape(1, E))

    sc_params = pltpu.CompilerParams(needs_layout_passes=False)
    mesh = plsc.VectorSubcoreMesh(core_axis_name="c", subcore_axis_name="s")
    dispatch = pl.kernel(
        functools.partial(_dispatch_body, N, NCH, CHA),
        mesh=mesh,
        compiler_params=sc_params,
        out_type=jax.ShapeDtypeStruct((SLOTS + 8, D // 2), jnp.int32),
        scratch_types=[
            pltpu.VMEM((NCH, CHA), jnp.int32),
            pltpu.VMEM((CHA, D // 2), jnp.int32),
            pltpu.VMEM((CHA, D // 2), jnp.int32),
            pltpu.SemaphoreType.DMA,
            pltpu.SemaphoreType.DMA,
            pltpu.SemaphoreType.DMA,
            pltpu.SemaphoreType.DMA,
        ],
    )
    xe = dispatch(xb16, dest_all)

    ffn = pl.pallas_call(
        functools.partial(_ffn_body, DH),
        grid=(E,),
        in_specs=[
            pl.BlockSpec((C, D // 2), lambda e: (e, 0)),
            pl.BlockSpec((1, 1, D), lambda e: (e, 0, 0)),
            pl.BlockSpec((1, 1, D), lambda e: (e, 0, 0)),
            pl.BlockSpec((1, D, 2 * DH), lambda e: (e, 0, 0)),
            pl.BlockSpec((1, 1, 2 * DH), lambda e: (e, 0, 0)),
            pl.BlockSpec((1, DH, D), lambda e: (e, 0, 0)),
            pl.BlockSpec((1, 1, D), lambda e: (e, 0, 0)),
        ],
        out_specs=pl.BlockSpec((C, D), lambda e: (e, 0)),
        out_shape=jax.ShapeDtypeStruct((SLOTS, D), jnp.float32),
    )
    oexp = ffn(xe, gamma.reshape(E, 1, D), beta.reshape(E, 1, D), W1,
               b1.reshape(E, 1, 2 * DH), W2, b2.reshape(E, 1, D))

    combine = pl.kernel(
        functools.partial(_combine_body, D, TOK, TCH),
        mesh=plsc.VectorSubcoreMesh(core_axis_name="c", subcore_axis_name="s"),
        compiler_params=sc_params,
        out_type=jax.ShapeDtypeStruct((N, D), jnp.float32),
        scratch_types=[
            pltpu.VMEM((TOK,), jnp.int32),
            pltpu.VMEM((TOK,), jnp.int32),
            pltpu.VMEM((TOK,), jnp.float32),
            pltpu.VMEM((TOK,), jnp.float32),
            pltpu.VMEM((TCH, D), jnp.float32),
            pltpu.VMEM((TCH, D), jnp.float32),
            pltpu.VMEM((TCH, D), jnp.float32),
            pltpu.VMEM((TCH, D), jnp.float32),
            pltpu.VMEM((TCH, D), jnp.float32),
            pltpu.VMEM((TCH, D), jnp.float32),
            pltpu.SemaphoreType.DMA,
            pltpu.SemaphoreType.DMA,
            pltpu.SemaphoreType.DMA,
            pltpu.SemaphoreType.DMA,
            pltpu.SemaphoreType.DMA,
            pltpu.SemaphoreType.DMA,
        ],
    )
    y = combine(oexp, s0, s1, w0, w1)
    return y.reshape(B, T, D)
